# Initial kernel scaffold; baseline (speedup 1.0000x reference)
import jax
import jax.numpy as jnp
from jax.experimental import pallas as pl


def kernel(x, support_embeddings, support_labels, temperature):
    raise NotImplementedError("write your pallas kernel here")


# speedup vs baseline: 8.1371x; 8.1371x over previous
import dataclasses
import functools

import jax
import jax.numpy as jnp
from jax import lax
from jax.experimental import pallas as pl
from jax.experimental.pallas import tpu as pltpu
from jax.experimental.pallas import tpu_sc as plsc

B = 4096
N = 100000
D = 64
K = 32
NCLS = 100
S = 2048
NCH = 49
NPAD = NCH * S
BQ = 256
INF = float("inf")


def _tc_select_body(x_ref, st_ref, t_ref, w_ref, idx_ref, dm_ref, m_ref):
    xb = x_ref[...]
    x2 = jnp.sum(xb * xb, axis=1, keepdims=True)
    laneS = lax.broadcasted_iota(jnp.int32, (BQ, S), 1)
    laneK = lax.broadcasted_iota(jnp.int32, (BQ, K), 1)

    def chunk_d2(c):
        sb = st_ref[c]
        prod = jnp.dot(xb, sb, preferred_element_type=jnp.float32)
        s2 = jnp.sum(sb * sb, axis=0, keepdims=True)
        d2 = x2 + s2 - 2.0 * prod
        d2 = jnp.maximum(d2, 1e-12)
        valid = (c * S + laneS) < N
        return jnp.where(valid, d2, INF)

    m_ref[...] = jnp.full((BQ, 128), INF, jnp.float32)

    def pass_a(c, _):
        d2 = chunk_d2(c)
        m = m_ref[...]
        for g in range(S // 128):
            m = jnp.minimum(m, d2[:, g * 128:(g + 1) * 128])
        m_ref[...] = m
        return 0

    lax.fori_loop(0, NCH, pass_a, 0)

    def kth(i, carry):
        mm, _ = carry
        mn = jnp.min(mm, axis=1, keepdims=True)
        mm = jnp.where(mm == mn, INF, mm)
        return mm, mn

    _, t0 = lax.fori_loop(0, K, kth, (m_ref[...], jnp.zeros((BQ, 1), jnp.float32)))

    def pass_b(c, carry):
        bd, bi = carry
        d2 = chunk_d2(c)
        dm_ref[...] = jnp.where(d2 <= t0, d2, INF)

        mn0 = jnp.min(dm_ref[...], axis=1, keepdims=True)
        cm0 = jnp.max(bd, axis=1, keepdims=True)
        go0 = jnp.any(mn0 < cm0)

        def cond(st):
            return st[4]

        def body(st):
            bd, bi, mn, _, _ = st
            dm = dm_ref[...]
            jl = jnp.min(jnp.where(dm == mn, laneS, S), axis=1, keepdims=True)
            cm = jnp.max(bd, axis=1, keepdims=True)
            acc = mn < cm
            pk = jnp.min(jnp.where(bd == cm, laneK, K), axis=1, keepdims=True)
            hit = (laneK == pk) & acc
            bd = jnp.where(hit, mn, bd)
            bi = jnp.where(hit, c * S + jl, bi)
            dm = jnp.where(laneS == jl, INF, dm)
            dm_ref[...] = dm
            mn2 = jnp.min(dm, axis=1, keepdims=True)
            cm2 = jnp.max(bd, axis=1, keepdims=True)
            return bd, bi, mn2, cm2, jnp.any(mn2 < cm2)

        bd, bi, _, _, _ = lax.while_loop(cond, body, (bd, bi, mn0, cm0, go0))
        return bd, bi

    bd0 = jnp.full((BQ, K), INF, jnp.float32)
    bi0 = jnp.zeros((BQ, K), jnp.int32)
    bd, bi = lax.fori_loop(0, NCH, pass_b, (bd0, bi0))

    d = jnp.sqrt(bd)
    temp = t_ref[0, 0]
    logits = -d / temp
    mx = jnp.max(logits, axis=1, keepdims=True)
    e = jnp.exp(logits - mx)
    w = e / jnp.sum(e, axis=1, keepdims=True)
    w_ref[...] = w
    idx_ref[...] = bi


def _tc_select(x, st3, temp11):
    return pl.pallas_call(
        _tc_select_body,
        grid=(B // BQ,),
        in_specs=[
            pl.BlockSpec((BQ, D), lambda i: (i, 0)),
            pl.BlockSpec((NCH, D, S), lambda i: (0, 0, 0)),
            pl.BlockSpec(memory_space=pltpu.SMEM),
        ],
        out_specs=[
            pl.BlockSpec((BQ, K), lambda i: (i, 0)),
            pl.BlockSpec((BQ, K), lambda i: (i, 0)),
        ],
        out_shape=[
            jax.ShapeDtypeStruct((B, K), jnp.float32),
            jax.ShapeDtypeStruct((B, K), jnp.int32),
        ],
        scratch_shapes=[
            pltpu.VMEM((BQ, S), jnp.float32),
            pltpu.VMEM((BQ, 128), jnp.float32),
        ],
        compiler_params=pltpu.CompilerParams(
            vmem_limit_bytes=100 * 1024 * 1024,
        ),
    )(x, st3, temp11)


def _sc_combine(w_flat, idx_flat, labels):
    info = plsc.get_sparse_core_info()
    nc, ns = info.num_cores, info.num_subcores
    nw = nc * ns
    rw = B // nw
    rb = 32
    nblk = rw // rb
    mesh = plsc.VectorSubcoreMesh(core_axis_name="c", subcore_axis_name="s")
    cp = pltpu.CompilerParams()
    if "needs_layout_passes" in pltpu.CompilerParams.__dataclass_fields__:
        cp = dataclasses.replace(cp, needs_layout_passes=False)

    @functools.partial(
        pl.kernel,
        mesh=mesh,
        compiler_params=cp,
        out_type=jax.ShapeDtypeStruct((B * NCLS,), jnp.float32),
        scratch_types=[
            pltpu.VMEM((N,), jnp.int32),
            pltpu.VMEM((rb * K,), jnp.int32),
            pltpu.VMEM((rb * K,), jnp.float32),
            pltpu.VMEM((rb * NCLS,), jnp.float32),
        ],
    )
    def sc_kernel(w_hbm, idx_hbm, lab_hbm, out_hbm, labv, idxv, wv, pv):
        wid = lax.axis_index("s") * nc + lax.axis_index("c")
        pltpu.sync_copy(lab_hbm, labv)
        zero16 = jnp.zeros((16,), jnp.float32)

        def do_block(blk, _):
            base = (wid * rw + blk * rb)

            pltpu.sync_copy(idx_hbm.at[pl.ds(base * K, rb * K)], idxv)
            pltpu.sync_copy(w_hbm.at[pl.ds(base * K, rb * K)], wv)

            def zero_row(i, _):
                pv[pl.ds(i * 16, 16)] = zero16
                return 0

            lax.fori_loop(0, (rb * NCLS) // 16, zero_row, 0)

            def do_row(r, _):
                for g in range(K // 16):
                    iv = idxv[pl.ds(r * K + g * 16, 16)]
                    lab = plsc.load_gather(labv, [iv])
                    wvec = wv[pl.ds(r * K + g * 16, 16)]
                    flat = lab + r * NCLS
                    plsc.addupdate_scatter(pv, [flat], wvec)
                return 0

            lax.fori_loop(0, rb, do_row, 0)
            pltpu.sync_copy(pv, out_hbm.at[pl.ds(base * NCLS, rb * NCLS)])
            return 0

        lax.fori_loop(0, nblk, do_block, 0)

    return sc_kernel(w_flat, idx_flat, labels)


def kernel(x, support_embeddings, support_labels, temperature):
    sp = jnp.pad(support_embeddings, ((0, NPAD - N), (0, 0)))
    st3 = sp.reshape(NCH, S, D).transpose(0, 2, 1)
    temp11 = jnp.reshape(temperature, (1, 1)).astype(jnp.float32)
    w, idx = _tc_select(x, st3, temp11)
    probs_flat = _sc_combine(w.reshape(-1), idx.reshape(-1), support_labels)
    return probs_flat.reshape(B, NCLS)
